# Initial kernel scaffold; baseline (speedup 1.0000x reference)
import jax
import jax.numpy as jnp
from jax.experimental import pallas as pl


def kernel(edge_index, node_emb, edge_emb, W_0, asrc_0, adst_0, We_0, aedge_0, b_0, W_1, asrc_1, adst_1, We_1, aedge_1, b_1, mlp_W1, mlp_b1, mlp_W2, mlp_b2):
    raise NotImplementedError("write your pallas kernel here")


# speedup vs baseline: 19.3917x; 19.3917x over previous
import functools

import numpy as np
import jax
import jax.numpy as jnp
from jax import lax
from jax.experimental import pallas as pl
from jax.experimental.pallas import tpu as pltpu
from jax.experimental.pallas import tpu_sc as plsc

N_RAW = 100000
E_RAW = 1600000
D_IN = 64
H = 16

NP = 100096
NW = 32
BLK = 128
GB = 16
NG = 25
EPW = NG * GB * BLK
EP = NW * EPW
NSTRIPE = NP // 16

_IDX16 = None


def _mm_body(x_ref, w_ref, o_ref):
    o_ref[...] = jnp.dot(x_ref[...], w_ref[...],
                         preferred_element_type=jnp.float32)


def _tc_matmul(x, w, block_rows):
    rows = x.shape[0]
    grid = rows // block_rows
    return pl.pallas_call(
        _mm_body,
        out_shape=jax.ShapeDtypeStruct((rows, w.shape[1]), jnp.float32),
        grid=(grid,),
        in_specs=[pl.BlockSpec((block_rows, x.shape[1]), lambda i: (i, 0)),
                  pl.BlockSpec(w.shape, lambda i: (0, 0))],
        out_specs=pl.BlockSpec((block_rows, w.shape[1]), lambda i: (i, 0)),
    )(x, w)


def _proj2_body(x_ref, wa_ref, wb_ref, a_ref, b_ref):
    x = x_ref[...]
    a_ref[...] = jnp.dot(x, wa_ref[...], preferred_element_type=jnp.float32)
    b_ref[...] = jnp.dot(x, wb_ref[...], preferred_element_type=jnp.float32)


def _tc_proj2(x, wa, wb):
    rows = x.shape[0]
    br = rows // 4
    return pl.pallas_call(
        _proj2_body,
        out_shape=(jax.ShapeDtypeStruct((rows, 128), jnp.float32),
                   jax.ShapeDtypeStruct((rows, 128), jnp.float32)),
        grid=(4,),
        in_specs=[pl.BlockSpec((br, x.shape[1]), lambda i: (i, 0)),
                  pl.BlockSpec(wa.shape, lambda i: (0, 0)),
                  pl.BlockSpec(wb.shape, lambda i: (0, 0))],
        out_specs=(pl.BlockSpec((br, 128), lambda i: (i, 0)),
                   pl.BlockSpec((br, 128), lambda i: (i, 0))),
    )(x, wa, wb)


def _nodeact_body(num_ref, den_ref, b_ref, w1_ref, wd_ref, xp_ref, d_ref):
    num = num_ref[0] + num_ref[1]
    x = jax.nn.gelu(num / (den_ref[...] + 1e-16) + b_ref[...])
    xp_ref[...] = jnp.dot(x, w1_ref[...], preferred_element_type=jnp.float32)
    d_ref[...] = jnp.dot(x, wd_ref[...], preferred_element_type=jnp.float32)


def _tc_node_activation(num2, den16, btile, bd_w, bd_wd):
    rows = NP // 8
    br = rows // 4
    return pl.pallas_call(
        _nodeact_body,
        out_shape=(jax.ShapeDtypeStruct((rows, 128), jnp.float32),
                   jax.ShapeDtypeStruct((rows, 128), jnp.float32)),
        grid=(4,),
        in_specs=[pl.BlockSpec((2, br, 128), lambda i: (0, i, 0)),
                  pl.BlockSpec((br, 128), lambda i: (i, 0)),
                  pl.BlockSpec((1, 128), lambda i: (0, 0)),
                  pl.BlockSpec((128, 128), lambda i: (0, 0)),
                  pl.BlockSpec((128, 128), lambda i: (0, 0))],
        out_specs=(pl.BlockSpec((br, 128), lambda i: (i, 0)),
                   pl.BlockSpec((br, 128), lambda i: (i, 0))),
    )(num2, den16, btile, bd_w, bd_wd)


def _uv_body(num_ref, den_ref, b_ref, wu_ref, wv_ref, bu_ref, u_ref, v_ref):
    num = num_ref[0] + num_ref[1]
    x = jax.nn.gelu(num / (den_ref[...] + 1e-16) + b_ref[...])
    u_ref[...] = jnp.dot(x, wu_ref[...],
                         preferred_element_type=jnp.float32) + bu_ref[...]
    v_ref[...] = jnp.dot(x, wv_ref[...], preferred_element_type=jnp.float32)


def _tc_uv(num2, den16, btile, bd_wu, bd_wv, b1tile):
    rows = NP // 8
    br = rows // 4
    return pl.pallas_call(
        _uv_body,
        out_shape=(jax.ShapeDtypeStruct((rows, 128), jnp.float32),
                   jax.ShapeDtypeStruct((rows, 128), jnp.float32)),
        grid=(4,),
        in_specs=[pl.BlockSpec((2, br, 128), lambda i: (0, i, 0)),
                  pl.BlockSpec((br, 128), lambda i: (i, 0)),
                  pl.BlockSpec((1, 128), lambda i: (0, 0)),
                  pl.BlockSpec((128, 128), lambda i: (0, 0)),
                  pl.BlockSpec((128, 128), lambda i: (0, 0)),
                  pl.BlockSpec((1, 128), lambda i: (0, 0))],
        out_specs=(pl.BlockSpec((br, 128), lambda i: (i, 0)),
                   pl.BlockSpec((br, 128), lambda i: (i, 0))),
    )(num2, den16, btile, bd_wu, bd_wv, b1tile)


_FULL15 = None


_GDN = lax.GatherDimensionNumbers(offset_dims=(), collapsed_slice_dims=(0,),
                                  start_index_map=(0,))


def _shuf(v, perm):
    return lax.gather(v, perm, _GDN, slice_sizes=(1,),
                      mode=lax.GatherScatterMode.PROMISE_IN_BOUNDS)


def _sumall(v):
    lanes = lax.iota(jnp.int32, 16)
    for s in (8, 4, 2, 1):
        v = v + _shuf(v, (lanes ^ s).reshape(16, 1))
    return v


def _make_layer_pass():
    mesh = plsc.VectorSubcoreMesh(core_axis_name="c", subcore_axis_name="s")

    @functools.partial(
        pl.kernel,
        compiler_params=pltpu.CompilerParams(use_tc_tiling_on_sc=False),
        out_type=(
            jax.ShapeDtypeStruct((2, NP, 16), jnp.float32),
            jax.ShapeDtypeStruct((2 * NP,), jnp.float32),
        ),
        mesh=mesh,
        scratch_types=dict(
            src_i=pltpu.VMEM((GB, BLK), jnp.int32),
            dst_i=pltpu.VMEM((GB, BLK), jnp.int32),
            didx=pltpu.VMEM((GB, BLK), jnp.int32),
            ee_v=pltpu.VMEM((BLK * 16,), jnp.float32),
            rows_v=pltpu.VMEM((BLK, 16), jnp.float32),
            d_v=pltpu.VMEM((BLK,), jnp.float32),
            ex_v=pltpu.VMEM((BLK,), jnp.float32),
            w_v=pltpu.VMEM((32,), jnp.float32),
            zero_v=pltpu.VMEM((136, 16), jnp.float32),
            acc=pltpu.VMEM_SHARED((NP, 16), jnp.float32),
            accs=pltpu.VMEM_SHARED((NP,), jnp.float32),
            sem=pltpu.SemaphoreType.DMA,
            sem2=pltpu.SemaphoreType.DMA,
        ),
    )
    def layer_pass(src2, dst2, ee1, tab, d1d, asrc_h, wc_h, out16, outS,
                   src_i, dst_i, didx, ee_v, rows_v, d_v, ex_v, w_v,
                   zero_v, acc, accs, sem, sem2):
        cid = lax.axis_index("c")
        sid = lax.axis_index("s")
        wid = cid * 16 + sid

        pltpu.sync_copy(asrc_h, w_v.at[pl.ds(0, 16)])
        pltpu.sync_copy(wc_h, w_v.at[pl.ds(16, 16)])
        asrc_v = w_v[pl.ds(0, 16)]
        wc_v = w_v[pl.ds(16, 16)]

        def zv(r, carry):
            zero_v[r] = jnp.zeros((16,), jnp.float32)
            return carry
        lax.fori_loop(0, 136, zv, 0)
        base_r = sid * NSTRIPE

        def zloop(i, carry):
            pltpu.sync_copy(zero_v, acc.at[pl.ds(base_r + i * 136, 136)])
            return carry
        lax.fori_loop(0, 46, zloop, 0)

        def zloop2(i, carry):
            pltpu.sync_copy(zero_v.at[0], accs.at[pl.ds(base_r + i * 16, 16)])
            return carry
        lax.fori_loop(0, NSTRIPE // 16, zloop2, 0)
        plsc.subcore_barrier()

        rbase = wid * (EPW // BLK)

        def grp(g, carry):
            gb = rbase + g * GB
            pltpu.sync_copy(src2.at[pl.ds(gb, GB)], src_i)
            pltpu.sync_copy(dst2.at[pl.ds(gb, GB)], dst_i)

            def sc16(t, carry2):
                r = t // (BLK // 16)
                q = (t % (BLK // 16)) * 16
                didx[r, pl.ds(q, 16)] = dst_i[r, pl.ds(q, 16)] * 16
                return carry2
            lax.fori_loop(0, GB * (BLK // 16), sc16, 0)

            def blk(b, carry2):
                eb = (gb + b) * BLK
                a1 = pltpu.async_copy(tab.at[src_i.at[b]], rows_v, sem)
                a2 = pltpu.async_copy(d1d.at[didx.at[b]], d_v, sem2)
                pltpu.sync_copy(ee1.at[pl.ds(eb * 16, BLK * 16)], ee_v)
                a1.wait()
                a2.wait()

                lanes = lax.iota(jnp.int32, 16)

                def edge16(jj, carry3):
                    dvec = d_v[pl.ds(jj * 16, 16)]
                    exacc = jnp.zeros((16,), jnp.float32)
                    for k in range(16):
                        j = jj * 16 + k
                        row = rows_v[j]
                        eer = ee_v[pl.ds(j * 16, 16)]
                        sb = _sumall(row * asrc_v)
                        cb = _sumall(eer * wc_v)
                        db = _shuf(dvec, (lanes * 0 + k).reshape(16, 1))
                        logit = sb + cb + db
                        logit = jnp.where(logit > 0, logit, 0.2 * logit)
                        exv = jnp.exp(logit)
                        rows_v[j] = row * exv
                        exacc = jnp.where(lanes == k, exv, exacc)
                    ex_v[pl.ds(jj * 16, 16)] = exacc
                    return carry3
                lax.fori_loop(0, BLK // 16, edge16, 0)

                pltpu.sync_copy(rows_v, acc.at[dst_i.at[b]], add=True)
                pltpu.sync_copy(ex_v, accs.at[dst_i.at[b]], add=True)
                return carry2
            lax.fori_loop(0, GB, blk, 0)
            return carry
        lax.fori_loop(0, NG, grp, 0)

        plsc.subcore_barrier()

        def dump(i, carry):
            pltpu.sync_copy(acc.at[pl.ds(base_r + i * 136, 136)],
                            out16.at[cid, pl.ds(base_r + i * 136, 136)])
            return carry
        lax.fori_loop(0, 46, dump, 0)
        pltpu.sync_copy(accs.at[pl.ds(base_r, NSTRIPE)],
                        outS.at[pl.ds(cid * NP + base_r, NSTRIPE)])

    return layer_pass


def _make_final_pass():
    mesh = plsc.VectorSubcoreMesh(core_axis_name="c", subcore_axis_name="s")

    @functools.partial(
        pl.kernel,
        compiler_params=pltpu.CompilerParams(use_tc_tiling_on_sc=False),
        out_type=jax.ShapeDtypeStruct((EP,), jnp.float32),
        mesh=mesh,
        scratch_types=dict(
            src_i=pltpu.VMEM((GB, BLK), jnp.int32),
            dst_i=pltpu.VMEM((GB, BLK), jnp.int32),
            u_v=pltpu.VMEM((BLK, 16), jnp.float32),
            v_v=pltpu.VMEM((BLK, 16), jnp.float32),
            g_v=pltpu.VMEM((BLK * 16,), jnp.float32),
            t_v=pltpu.VMEM((BLK,), jnp.float32),
            p_v=pltpu.VMEM((BLK,), jnp.float32),
            w_v=pltpu.VMEM((32,), jnp.float32),
            sem=pltpu.SemaphoreType.DMA,
            sem2=pltpu.SemaphoreType.DMA,
        ),
    )
    def final_pass(src2, dst2, g1, utab, vtab, w2_h, b2_h, outp,
                   src_i, dst_i, u_v, v_v, g_v, t_v, p_v, w_v, sem, sem2):
        cid = lax.axis_index("c")
        sid = lax.axis_index("s")
        wid = cid * 16 + sid

        pltpu.sync_copy(w2_h, w_v.at[pl.ds(0, 16)])
        pltpu.sync_copy(b2_h, w_v.at[pl.ds(16, 16)])
        w2_v = w_v[pl.ds(0, 16)]
        b2_v = w_v[pl.ds(16, 16)]

        rbase = wid * (EPW // BLK)

        def grp(g, carry):
            gb = rbase + g * GB
            pltpu.sync_copy(src2.at[pl.ds(gb, GB)], src_i)
            pltpu.sync_copy(dst2.at[pl.ds(gb, GB)], dst_i)

            def blk(b, carry2):
                eb = (gb + b) * BLK
                a1 = pltpu.async_copy(utab.at[src_i.at[b]], u_v, sem)
                a2 = pltpu.async_copy(vtab.at[dst_i.at[b]], v_v, sem2)
                pltpu.sync_copy(g1.at[pl.ds(eb * 16, BLK * 16)], g_v)
                a1.wait()
                a2.wait()

                lanes = lax.iota(jnp.int32, 16)

                def edge16(jj, carry3):
                    tacc = jnp.zeros((16,), jnp.float32)
                    for k in range(16):
                        j = jj * 16 + k
                        h = u_v[j] + v_v[j] + g_v[pl.ds(j * 16, 16)]
                        h = jnp.maximum(h, 0.0)
                        tb = _sumall(h * w2_v)
                        tacc = jnp.where(lanes == k, tb, tacc)
                    t = tacc + b2_v
                    p_v[pl.ds(jj * 16, 16)] = 1.0 / (1.0 + jnp.exp(-t))
                    return carry3
                lax.fori_loop(0, BLK // 16, edge16, 0)

                pltpu.sync_copy(p_v, outp.at[pl.ds(eb, BLK)])
                return carry2
            lax.fori_loop(0, GB, blk, 0)
            return carry
        lax.fori_loop(0, NG, grp, 0)

    return final_pass


def kernel(edge_index, node_emb, edge_emb, W_0, asrc_0, adst_0, We_0, aedge_0,
           b_0, W_1, asrc_1, adst_1, We_1, aedge_1, b_1, mlp_W1, mlp_b1,
           mlp_W2, mlp_b2):
    f32 = jnp.float32
    src = edge_index[0]
    dst = edge_index[1]
    pad_e = EP - E_RAW
    src_p = jnp.concatenate([src, jnp.zeros((pad_e,), jnp.int32)])
    dst_p = jnp.concatenate([dst, jnp.full((pad_e,), N_RAW, jnp.int32)])
    src2 = src_p.reshape(EP // BLK, BLK)
    dst2 = dst_p.reshape(EP // BLK, BLK)

    ee_p = jnp.pad(edge_emb.astype(f32), ((0, pad_e), (0, 0)))
    ee1 = ee_p.reshape(EP * H)
    ee8 = ee_p.reshape(EP * H // 128, 128)

    ne_p = jnp.pad(node_emb.astype(f32), ((0, NP - N_RAW), (0, 0)))

    eye8 = jnp.eye(8, dtype=f32)
    ones16 = jnp.ones((1, 16), f32)
    ne8 = ne_p.reshape(NP // 8, 8 * D_IN)

    Md0 = adst_0[:, None] @ ones16
    K_xp = jnp.kron(eye8, W_0.astype(f32))
    K_d = jnp.kron(eye8, (W_0 @ Md0).astype(f32))
    xp0_8, d0_8 = _tc_proj2(ne8, K_xp, K_d)
    tab0 = xp0_8.reshape(NP, 16)
    d0_1d = d0_8.reshape(NP * 16)

    Wc = mlp_W1[2 * H:3 * H, :]
    BD8_Wc = jnp.kron(eye8, Wc)
    g8 = _tc_matmul(ee8, BD8_Wc, 2048)
    g1 = g8.reshape(EP * H)

    wc0 = We_0 @ aedge_0
    lp = _make_layer_pass()
    out16_0, outS_0 = lp(src2, dst2, ee1, tab0, d0_1d,
                         asrc_0.astype(f32), wc0.astype(f32))

    num2_0 = out16_0.reshape(2, NP // 8, 128)
    den_0 = outS_0[:NP] + outS_0[NP:]
    den16_0 = jnp.repeat(den_0, H).reshape(NP // 8, 128)
    b0tile = jnp.tile(b_0.astype(f32), 8)[None, :]
    BD8_W1 = jnp.kron(eye8, W_1.astype(f32))
    Md1 = adst_1[:, None] @ ones16
    BD8_W1Md = jnp.kron(eye8, (W_1 @ Md1).astype(f32))
    xp1_8, d1_8 = _tc_node_activation(num2_0, den16_0, b0tile,
                                      BD8_W1, BD8_W1Md)
    tab1 = xp1_8.reshape(NP, 16)
    d1_1d = d1_8.reshape(NP * 16)

    wc1 = We_1 @ aedge_1
    out16_1, outS_1 = lp(src2, dst2, ee1, tab1, d1_1d,
                         asrc_1.astype(f32), wc1.astype(f32))

    num2_1 = out16_1.reshape(2, NP // 8, 128)
    den_1 = outS_1[:NP] + outS_1[NP:]
    den16_1 = jnp.repeat(den_1, H).reshape(NP // 8, 128)
    b1gtile = jnp.tile(b_1.astype(f32), 8)[None, :]
    Wa = mlp_W1[:H, :]
    Wb = mlp_W1[H:2 * H, :]
    BD8_Wa = jnp.kron(eye8, Wa.astype(f32))
    BD8_Wb = jnp.kron(eye8, Wb.astype(f32))
    b1tile = jnp.tile(mlp_b1.astype(f32), 8)[None, :]
    u8, v8 = _tc_uv(num2_1, den16_1, b1gtile, BD8_Wa, BD8_Wb, b1tile)
    utab = u8.reshape(NP, 16)
    vtab = v8.reshape(NP, 16)

    fp = _make_final_pass()
    w2_h = mlp_W2[:, 0].astype(f32)
    b2_h = jnp.full((16,), mlp_b2[0], f32)
    p = fp(src2, dst2, g1, utab, vtab, w2_h, b2_h)
    return p[:E_RAW]


# speedup vs baseline: 37.3136x; 1.9242x over previous
import functools

import numpy as np
import jax
import jax.numpy as jnp
from jax import lax
from jax.experimental import pallas as pl
from jax.experimental.pallas import tpu as pltpu
from jax.experimental.pallas import tpu_sc as plsc

N_RAW = 100000
E_RAW = 1600000
D_IN = 64
H = 16

NP = 100096
NW = 32
BLK = 128
GB = 16
NG = 25
EPW = NG * GB * BLK
EP = NW * EPW
NSTRIPE = NP // 16


def _proj3_body(x_ref, wa_ref, wb_ref, wc_ref, a_ref, b_ref, c_ref):
    x = x_ref[...]
    a_ref[...] = jnp.dot(x, wa_ref[...], preferred_element_type=jnp.float32)
    b_ref[...] = jnp.dot(x, wb_ref[...], preferred_element_type=jnp.float32)
    c_ref[...] = jnp.dot(x, wc_ref[...], preferred_element_type=jnp.float32)


def _tc_proj3(x, wa, wb, wc, grid):
    rows = x.shape[0]
    br = rows // grid
    out = jax.ShapeDtypeStruct((rows, 128), jnp.float32)
    wspec = pl.BlockSpec(wa.shape, lambda i: (0, 0))
    bspec = pl.BlockSpec((br, 128), lambda i: (i, 0))
    return pl.pallas_call(
        _proj3_body,
        out_shape=(out, out, out),
        grid=(grid,),
        in_specs=[pl.BlockSpec((br, x.shape[1]), lambda i: (i, 0)),
                  wspec, wspec, wspec],
        out_specs=(bspec, bspec, bspec),
    )(x, wa, wb, wc)


def _nodeact_body(num_ref, den_ref, b_ref, w1_ref, wd_ref, ws_ref,
                  xp_ref, d_ref, s_ref):
    num = num_ref[0] + num_ref[1]
    x = jax.nn.gelu(num / (den_ref[...] + 1e-16) + b_ref[...])
    xp_ref[...] = jnp.dot(x, w1_ref[...], preferred_element_type=jnp.float32)
    d_ref[...] = jnp.dot(x, wd_ref[...], preferred_element_type=jnp.float32)
    s_ref[...] = jnp.dot(x, ws_ref[...], preferred_element_type=jnp.float32)


def _tc_node_activation(num2, den16, btile, bd_w, bd_wd, bd_ws):
    rows = NP // 8
    br = rows // 4
    out = jax.ShapeDtypeStruct((rows, 128), jnp.float32)
    wspec = pl.BlockSpec((128, 128), lambda i: (0, 0))
    bspec = pl.BlockSpec((br, 128), lambda i: (i, 0))
    return pl.pallas_call(
        _nodeact_body,
        out_shape=(out, out, out),
        grid=(4,),
        in_specs=[pl.BlockSpec((2, br, 128), lambda i: (0, i, 0)),
                  bspec,
                  pl.BlockSpec((1, 128), lambda i: (0, 0)),
                  wspec, wspec, wspec],
        out_specs=(bspec, bspec, bspec),
    )(num2, den16, btile, bd_w, bd_wd, bd_ws)


def _uv_body(num_ref, den_ref, b_ref, wu_ref, wv_ref, bu_ref, u_ref, v_ref):
    num = num_ref[0] + num_ref[1]
    x = jax.nn.gelu(num / (den_ref[...] + 1e-16) + b_ref[...])
    u_ref[...] = jnp.dot(x, wu_ref[...],
                         preferred_element_type=jnp.float32) + bu_ref[...]
    v_ref[...] = jnp.dot(x, wv_ref[...], preferred_element_type=jnp.float32)


def _tc_uv(num2, den16, btile, bd_wu, bd_wv, b1tile):
    rows = NP // 8
    br = rows // 4
    out = jax.ShapeDtypeStruct((rows, 128), jnp.float32)
    wspec = pl.BlockSpec((128, 128), lambda i: (0, 0))
    bspec = pl.BlockSpec((br, 128), lambda i: (i, 0))
    return pl.pallas_call(
        _uv_body,
        out_shape=(out, out),
        grid=(4,),
        in_specs=[pl.BlockSpec((2, br, 128), lambda i: (0, i, 0)),
                  bspec,
                  pl.BlockSpec((1, 128), lambda i: (0, 0)),
                  wspec, wspec,
                  pl.BlockSpec((1, 128), lambda i: (0, 0))],
        out_specs=(bspec, bspec),
    )(num2, den16, btile, bd_wu, bd_wv, b1tile)


_GDN = lax.GatherDimensionNumbers(offset_dims=(), collapsed_slice_dims=(0,),
                                  start_index_map=(0,))


def _shuf(v, perm):
    return lax.gather(v, perm, _GDN, slice_sizes=(1,),
                      mode=lax.GatherScatterMode.PROMISE_IN_BOUNDS)


def _sumall(v):
    lanes = lax.iota(jnp.int32, 16)
    for s in (8, 4, 2, 1):
        v = v + _shuf(v, (lanes ^ s).reshape(16, 1))
    return v


def _make_layer_pass():
    mesh = plsc.VectorSubcoreMesh(core_axis_name="c", subcore_axis_name="s")

    @functools.partial(
        pl.kernel,
        compiler_params=pltpu.CompilerParams(use_tc_tiling_on_sc=False),
        out_type=(
            jax.ShapeDtypeStruct((2, NP, 16), jnp.float32),
            jax.ShapeDtypeStruct((2 * NP,), jnp.float32),
        ),
        mesh=mesh,
        scratch_types=dict(
            src_i=pltpu.VMEM((GB, BLK), jnp.int32),
            dst_i=pltpu.VMEM((GB, BLK), jnp.int32),
            sidx=pltpu.VMEM((GB, BLK), jnp.int32),
            didx=pltpu.VMEM((GB, BLK), jnp.int32),
            cidx=pltpu.VMEM((GB, BLK), jnp.int32),
            rows0=pltpu.VMEM((BLK, 16), jnp.float32),
            rows1=pltpu.VMEM((BLK, 16), jnp.float32),
            s0=pltpu.VMEM((BLK,), jnp.float32),
            s1=pltpu.VMEM((BLK,), jnp.float32),
            d0=pltpu.VMEM((BLK,), jnp.float32),
            d1=pltpu.VMEM((BLK,), jnp.float32),
            c0=pltpu.VMEM((BLK,), jnp.float32),
            c1=pltpu.VMEM((BLK,), jnp.float32),
            ex0=pltpu.VMEM((BLK,), jnp.float32),
            ex1=pltpu.VMEM((BLK,), jnp.float32),
            zero_v=pltpu.VMEM((136, 16), jnp.float32),
            acc=pltpu.VMEM_SHARED((NP, 16), jnp.float32),
            accs=pltpu.VMEM_SHARED((NP,), jnp.float32),
            semg=pltpu.SemaphoreType.DMA,
            sems=pltpu.SemaphoreType.DMA,
        ),
    )
    def layer_pass(src2, dst2, tab, s1d, d1d, c1d, out16, outS,
                   src_i, dst_i, sidx, didx, cidx, rows0, rows1,
                   s0, s1, d0, d1, c0, c1, ex0, ex1,
                   zero_v, acc, accs, semg, sems):
        cid = lax.axis_index("c")
        sid = lax.axis_index("s")
        wid = cid * 16 + sid

        def zv(r, carry):
            zero_v[r] = jnp.zeros((16,), jnp.float32)
            return carry
        lax.fori_loop(0, 136, zv, 0)
        base_r = sid * NSTRIPE

        def zloop(i, carry):
            pltpu.sync_copy(zero_v, acc.at[pl.ds(base_r + i * 136, 136)])
            return carry
        lax.fori_loop(0, 46, zloop, 0)

        def zloop2(i, carry):
            pltpu.sync_copy(zero_v.at[0], accs.at[pl.ds(base_r + i * 16, 16)])
            return carry
        lax.fori_loop(0, NSTRIPE // 16, zloop2, 0)
        plsc.subcore_barrier()

        rbase = wid * (EPW // BLK)
        lanes = lax.iota(jnp.int32, 16)

        def grp(g, carry):
            gb = rbase + g * GB
            pltpu.sync_copy(src2.at[pl.ds(gb, GB)], src_i)
            pltpu.sync_copy(dst2.at[pl.ds(gb, GB)], dst_i)

            def sc16(t, carry2):
                r = t // (BLK // 16)
                q = (t % (BLK // 16)) * 16
                sidx[r, pl.ds(q, 16)] = src_i[r, pl.ds(q, 16)] * 16
                didx[r, pl.ds(q, 16)] = dst_i[r, pl.ds(q, 16)] * 16
                cidx[r, pl.ds(q, 16)] = ((gb + r) * BLK + q + lanes) * 16
                return carry2
            lax.fori_loop(0, GB * (BLK // 16), sc16, 0)

            def pair(q, carry2):
                b0 = 2 * q
                b1 = 2 * q + 1
                bufs = ((rows0, s0, d0, c0, ex0, b0),
                        (rows1, s1, d1, c1, ex1, b1))
                handles = []
                for rows_v, s_v, d_v, c_v, ex_v, b in bufs:
                    handles.append((
                        pltpu.async_copy(tab.at[src_i.at[b]], rows_v, semg),
                        pltpu.async_copy(s1d.at[sidx.at[b]], s_v, semg),
                        pltpu.async_copy(d1d.at[didx.at[b]], d_v, semg),
                        pltpu.async_copy(c1d.at[cidx.at[b]], c_v, semg),
                    ))
                scat = []
                for (rows_v, s_v, d_v, c_v, ex_v, b), hs in zip(bufs, handles):
                    for hh in hs:
                        hh.wait()

                    def edge16(jj, carry3, rows_v=rows_v, s_v=s_v, d_v=d_v,
                               c_v=c_v, ex_v=ex_v):
                        logit = s_v[pl.ds(jj * 16, 16)] \
                            + d_v[pl.ds(jj * 16, 16)] \
                            + c_v[pl.ds(jj * 16, 16)]
                        logit = jnp.where(logit > 0, logit, 0.2 * logit)
                        exv = jnp.exp(logit)
                        ex_v[pl.ds(jj * 16, 16)] = exv
                        for k in range(16):
                            j = jj * 16 + k
                            exb = _shuf(exv, (lanes * 0 + k).reshape(16, 1))
                            rows_v[j] = rows_v[j] * exb
                        return carry3
                    lax.fori_loop(0, BLK // 16, edge16, 0)

                    scat.append(pltpu.async_copy(
                        rows_v, acc.at[dst_i.at[b]], sems, add=True))
                    scat.append(pltpu.async_copy(
                        ex_v, accs.at[dst_i.at[b]], sems, add=True))
                for hh in scat:
                    hh.wait()
                return carry2
            lax.fori_loop(0, GB // 2, pair, 0)
            return carry
        lax.fori_loop(0, NG, grp, 0)

        plsc.subcore_barrier()

        def dump(i, carry):
            pltpu.sync_copy(acc.at[pl.ds(base_r + i * 136, 136)],
                            out16.at[cid, pl.ds(base_r + i * 136, 136)])
            return carry
        lax.fori_loop(0, 46, dump, 0)
        pltpu.sync_copy(accs.at[pl.ds(base_r, NSTRIPE)],
                        outS.at[pl.ds(cid * NP + base_r, NSTRIPE)])

    return layer_pass


def _make_final_pass():
    mesh = plsc.VectorSubcoreMesh(core_axis_name="c", subcore_axis_name="s")

    @functools.partial(
        pl.kernel,
        compiler_params=pltpu.CompilerParams(use_tc_tiling_on_sc=False),
        out_type=jax.ShapeDtypeStruct((EP,), jnp.float32),
        mesh=mesh,
        scratch_types=dict(
            src_i=pltpu.VMEM((GB, BLK), jnp.int32),
            dst_i=pltpu.VMEM((GB, BLK), jnp.int32),
            u0=pltpu.VMEM((BLK, 16), jnp.float32),
            u1=pltpu.VMEM((BLK, 16), jnp.float32),
            v0=pltpu.VMEM((BLK, 16), jnp.float32),
            v1=pltpu.VMEM((BLK, 16), jnp.float32),
            g0=pltpu.VMEM((BLK * 16,), jnp.float32),
            g1v=pltpu.VMEM((BLK * 16,), jnp.float32),
            p_v=pltpu.VMEM((2 * BLK,), jnp.float32),
            w_v=pltpu.VMEM((32,), jnp.float32),
            semg=pltpu.SemaphoreType.DMA,
            semp=pltpu.SemaphoreType.DMA,
        ),
    )
    def final_pass(src2, dst2, g1, utab, vtab, w2_h, b2_h, outp,
                   src_i, dst_i, u0, u1, v0, v1, g0, g1v, p_v, w_v,
                   semg, semp):
        cid = lax.axis_index("c")
        sid = lax.axis_index("s")
        wid = cid * 16 + sid

        pltpu.sync_copy(w2_h, w_v.at[pl.ds(0, 16)])
        pltpu.sync_copy(b2_h, w_v.at[pl.ds(16, 16)])
        w2_v = w_v[pl.ds(0, 16)]
        b2_v = w_v[pl.ds(16, 16)]

        rbase = wid * (EPW // BLK)
        lanes = lax.iota(jnp.int32, 16)

        def grp(g, carry):
            gb = rbase + g * GB
            pltpu.sync_copy(src2.at[pl.ds(gb, GB)], src_i)
            pltpu.sync_copy(dst2.at[pl.ds(gb, GB)], dst_i)

            def pair(q, carry2):
                b0 = 2 * q
                b1 = 2 * q + 1
                eb0 = (gb + b0) * BLK
                bufs = ((u0, v0, g0, 0, b0), (u1, v1, g1v, BLK, b1))
                handles = []
                for u_v, v_v, g_v, poff, b in bufs:
                    handles.append((
                        pltpu.async_copy(utab.at[src_i.at[b]], u_v, semg),
                        pltpu.async_copy(vtab.at[dst_i.at[b]], v_v, semg),
                    ))
                    pltpu.sync_copy(
                        g1.at[pl.ds((gb + b) * BLK * 16, BLK * 16)], g_v)
                for (u_v, v_v, g_v, poff, b), hs in zip(bufs, handles):
                    for hh in hs:
                        hh.wait()

                    def edge16(jj, carry3, u_v=u_v, v_v=v_v, g_v=g_v,
                               poff=poff):
                        tacc = jnp.zeros((16,), jnp.float32)
                        for k in range(16):
                            j = jj * 16 + k
                            h = u_v[j] + v_v[j] + g_v[pl.ds(j * 16, 16)]
                            h = jnp.maximum(h, 0.0)
                            tb = _sumall(h * w2_v)
                            tacc = jnp.where(lanes == k, tb, tacc)
                        t = tacc + b2_v
                        p_v[pl.ds(poff + jj * 16, 16)] = \
                            1.0 / (1.0 + jnp.exp(-t))
                        return carry3
                    lax.fori_loop(0, BLK // 16, edge16, 0)
                pltpu.async_copy(p_v, outp.at[pl.ds(eb0, 2 * BLK)],
                                 semp).wait()
                return carry2
            lax.fori_loop(0, GB // 2, pair, 0)
            return carry
        lax.fori_loop(0, NG, grp, 0)

    return final_pass


def kernel(edge_index, node_emb, edge_emb, W_0, asrc_0, adst_0, We_0, aedge_0,
           b_0, W_1, asrc_1, adst_1, We_1, aedge_1, b_1, mlp_W1, mlp_b1,
           mlp_W2, mlp_b2):
    f32 = jnp.float32
    src = edge_index[0]
    dst = edge_index[1]
    pad_e = EP - E_RAW
    pad_ar = np.arange(pad_e, dtype=np.int32)
    src_p = jnp.concatenate([src, jnp.asarray(pad_ar % N_RAW)])
    dst_p = jnp.concatenate([dst, jnp.asarray(N_RAW + pad_ar % (NP - N_RAW))])
    src2 = src_p.reshape(EP // BLK, BLK)
    dst2 = dst_p.reshape(EP // BLK, BLK)

    ee_p = jnp.pad(edge_emb.astype(f32), ((0, pad_e), (0, 0)))
    ee8 = ee_p.reshape(EP * H // 128, 128)

    ne_p = jnp.pad(node_emb.astype(f32), ((0, NP - N_RAW), (0, 0)))
    ne8 = ne_p.reshape(NP // 8, 8 * D_IN)

    eye8 = jnp.eye(8, dtype=f32)
    ones16 = jnp.ones((1, 16), f32)

    def rep(vec):
        return vec.astype(f32)[:, None] @ ones16

    K_xp = jnp.kron(eye8, W_0.astype(f32))
    K_d = jnp.kron(eye8, W_0.astype(f32) @ rep(adst_0))
    K_s = jnp.kron(eye8, W_0.astype(f32) @ rep(asrc_0))
    xp0_8, d0_8, s0_8 = _tc_proj3(ne8, K_xp, K_d, K_s, 4)
    tab0 = xp0_8.reshape(NP, 16)
    d0_1d = d0_8.reshape(NP * 16)
    s0_1d = s0_8.reshape(NP * 16)

    Wc = mlp_W1[2 * H:3 * H, :].astype(f32)
    BD_g = jnp.kron(eye8, Wc)
    BD_c0 = jnp.kron(eye8, rep(We_0 @ aedge_0))
    BD_c1 = jnp.kron(eye8, rep(We_1 @ aedge_1))
    g8, c0_8, c1_8 = _tc_proj3(ee8, BD_g, BD_c0, BD_c1, 25)
    g1 = g8.reshape(EP * H)
    c0_1d = c0_8.reshape(EP * H)
    c1_1d = c1_8.reshape(EP * H)

    lp = _make_layer_pass()
    out16_0, outS_0 = lp(src2, dst2, tab0, s0_1d, d0_1d, c0_1d)

    num2_0 = out16_0.reshape(2, NP // 8, 128)
    den_0 = outS_0[:NP] + outS_0[NP:]
    den16_0 = jnp.repeat(den_0, H).reshape(NP // 8, 128)
    b0tile = jnp.tile(b_0.astype(f32), 8)[None, :]
    W1f = W_1.astype(f32)
    BD8_W1 = jnp.kron(eye8, W1f)
    BD8_W1Md = jnp.kron(eye8, W1f @ rep(adst_1))
    BD8_W1Ms = jnp.kron(eye8, W1f @ rep(asrc_1))
    xp1_8, d1_8, s1_8 = _tc_node_activation(num2_0, den16_0, b0tile,
                                            BD8_W1, BD8_W1Md, BD8_W1Ms)
    tab1 = xp1_8.reshape(NP, 16)
    d1_1d = d1_8.reshape(NP * 16)
    s1_1d = s1_8.reshape(NP * 16)

    out16_1, outS_1 = lp(src2, dst2, tab1, s1_1d, d1_1d, c1_1d)

    num2_1 = out16_1.reshape(2, NP // 8, 128)
    den_1 = outS_1[:NP] + outS_1[NP:]
    den16_1 = jnp.repeat(den_1, H).reshape(NP // 8, 128)
    b1gtile = jnp.tile(b_1.astype(f32), 8)[None, :]
    BD8_Wa = jnp.kron(eye8, mlp_W1[:H, :].astype(f32))
    BD8_Wb = jnp.kron(eye8, mlp_W1[H:2 * H, :].astype(f32))
    b1tile = jnp.tile(mlp_b1.astype(f32), 8)[None, :]
    u8, v8 = _tc_uv(num2_1, den16_1, b1gtile, BD8_Wa, BD8_Wb, b1tile)
    utab = u8.reshape(NP, 16)
    vtab = v8.reshape(NP, 16)

    fp = _make_final_pass()
    w2_h = mlp_W2[:, 0].astype(f32)
    b2_h = jnp.full((16,), mlp_b2[0], f32)
    p = fp(src2, dst2, g1, utab, vtab, w2_h, b2_h)
    return p[:E_RAW]


# speedup vs baseline: 45.7731x; 1.2267x over previous
import functools

import numpy as np
import jax
import jax.numpy as jnp
from jax import lax
from jax.experimental import pallas as pl
from jax.experimental.pallas import tpu as pltpu
from jax.experimental.pallas import tpu_sc as plsc

N_RAW = 100000
E_RAW = 1600000
D_IN = 64
H = 16

NP = 100096
NW = 32
BLK = 128
GB = 16
NG = 25
EPW = NG * GB * BLK
EP = NW * EPW
NSTRIPE = NP // 16
NB_L = 4
NB_F = 8
ER = E_RAW * H // 128


def _proj3_body(x_ref, wa_ref, wb_ref, wc_ref, a_ref, b_ref, c_ref):
    x = x_ref[...]
    a_ref[...] = jnp.dot(x, wa_ref[...], preferred_element_type=jnp.float32)
    b_ref[...] = jnp.dot(x, wb_ref[...], preferred_element_type=jnp.float32)
    c_ref[...] = jnp.dot(x, wc_ref[...], preferred_element_type=jnp.float32)


def _tc_proj3(x, wa, wb, wc, grid):
    rows = x.shape[0]
    br = rows // grid
    out = jax.ShapeDtypeStruct((rows, 128), jnp.float32)
    wspec = pl.BlockSpec(wa.shape, lambda i: (0, 0))
    bspec = pl.BlockSpec((br, 128), lambda i: (i, 0))
    return pl.pallas_call(
        _proj3_body,
        out_shape=(out, out, out),
        grid=(grid,),
        in_specs=[pl.BlockSpec((br, x.shape[1]), lambda i: (i, 0)),
                  wspec, wspec, wspec],
        out_specs=(bspec, bspec, bspec),
    )(x, wa, wb, wc)


def _nodeact_body(num_ref, den_ref, b_ref, w1_ref, wd_ref, ws_ref,
                  xp_ref, d_ref, s_ref):
    num = num_ref[0] + num_ref[1]
    x = jax.nn.gelu(num / (den_ref[...] + 1e-16) + b_ref[...])
    xp_ref[...] = jnp.dot(x, w1_ref[...], preferred_element_type=jnp.float32)
    d_ref[...] = jnp.dot(x, wd_ref[...], preferred_element_type=jnp.float32)
    s_ref[...] = jnp.dot(x, ws_ref[...], preferred_element_type=jnp.float32)


def _tc_node_activation(num2, den16, btile, bd_w, bd_wd, bd_ws):
    rows = NP // 8
    br = rows // 4
    out = jax.ShapeDtypeStruct((rows, 128), jnp.float32)
    wspec = pl.BlockSpec((128, 128), lambda i: (0, 0))
    bspec = pl.BlockSpec((br, 128), lambda i: (i, 0))
    return pl.pallas_call(
        _nodeact_body,
        out_shape=(out, out, out),
        grid=(4,),
        in_specs=[pl.BlockSpec((2, br, 128), lambda i: (0, i, 0)),
                  bspec,
                  pl.BlockSpec((1, 128), lambda i: (0, 0)),
                  wspec, wspec, wspec],
        out_specs=(bspec, bspec, bspec),
    )(num2, den16, btile, bd_w, bd_wd, bd_ws)


def _uv_body(num_ref, den_ref, b_ref, wu_ref, wv_ref, bu_ref, u_ref, v_ref):
    num = num_ref[0] + num_ref[1]
    x = jax.nn.gelu(num / (den_ref[...] + 1e-16) + b_ref[...])
    u_ref[...] = jnp.dot(x, wu_ref[...],
                         preferred_element_type=jnp.float32) + bu_ref[...]
    v_ref[...] = jnp.dot(x, wv_ref[...], preferred_element_type=jnp.float32)


def _tc_uv(num2, den16, btile, bd_wu, bd_wv, b1tile):
    rows = NP // 8
    br = rows // 4
    out = jax.ShapeDtypeStruct((rows, 128), jnp.float32)
    wspec = pl.BlockSpec((128, 128), lambda i: (0, 0))
    bspec = pl.BlockSpec((br, 128), lambda i: (i, 0))
    return pl.pallas_call(
        _uv_body,
        out_shape=(out, out),
        grid=(4,),
        in_specs=[pl.BlockSpec((2, br, 128), lambda i: (0, i, 0)),
                  bspec,
                  pl.BlockSpec((1, 128), lambda i: (0, 0)),
                  wspec, wspec,
                  pl.BlockSpec((1, 128), lambda i: (0, 0))],
        out_specs=(bspec, bspec),
    )(num2, den16, btile, bd_wu, bd_wv, b1tile)


_GDN = lax.GatherDimensionNumbers(offset_dims=(), collapsed_slice_dims=(0,),
                                  start_index_map=(0,))


def _shuf(v, perm):
    return lax.gather(v, perm, _GDN, slice_sizes=(1,),
                      mode=lax.GatherScatterMode.PROMISE_IN_BOUNDS)


def _sumall(v):
    lanes = lax.iota(jnp.int32, 16)
    for s in (8, 4, 2, 1):
        v = v + _shuf(v, (lanes ^ s).reshape(16, 1))
    return v


def _make_layer_pass():
    mesh = plsc.VectorSubcoreMesh(core_axis_name="c", subcore_axis_name="s")

    @functools.partial(
        pl.kernel,
        compiler_params=pltpu.CompilerParams(use_tc_tiling_on_sc=False),
        out_type=(
            jax.ShapeDtypeStruct((2, NP, 16), jnp.float32),
            jax.ShapeDtypeStruct((2 * NP,), jnp.float32),
        ),
        mesh=mesh,
        scratch_types=dict(
            src_i=pltpu.VMEM((GB, BLK), jnp.int32),
            dst_i=pltpu.VMEM((GB, BLK), jnp.int32),
            sidx=pltpu.VMEM((NB_L, BLK), jnp.int32),
            didx=pltpu.VMEM((NB_L, BLK), jnp.int32),
            cidx=pltpu.VMEM((NB_L, BLK), jnp.int32),
            rows_v=pltpu.VMEM((NB_L * BLK, 16), jnp.float32),
            s_v=pltpu.VMEM((NB_L * BLK,), jnp.float32),
            d_v=pltpu.VMEM((NB_L * BLK,), jnp.float32),
            c_v=pltpu.VMEM((NB_L * BLK,), jnp.float32),
            ex_v=pltpu.VMEM((NB_L * BLK,), jnp.float32),
            zero_v=pltpu.VMEM((68, 16), jnp.float32),
            acc=pltpu.VMEM_SHARED((NP, 16), jnp.float32),
            accs=pltpu.VMEM_SHARED((NP,), jnp.float32),
            semg=pltpu.SemaphoreType.DMA,
            sems=pltpu.SemaphoreType.DMA,
        ),
    )
    def layer_pass(src2, dst2, tab, s1d, d1d, c1d, out16, outS,
                   src_i, dst_i, sidx, didx, cidx, rows_v, s_v, d_v, c_v,
                   ex_v, zero_v, acc, accs, semg, sems):
        cid = lax.axis_index("c")
        sid = lax.axis_index("s")
        wid = cid * 16 + sid

        def zv(r, carry):
            zero_v[r] = jnp.zeros((16,), jnp.float32)
            return carry
        lax.fori_loop(0, 68, zv, 0)
        base_r = sid * NSTRIPE

        def zloop(i, carry):
            pltpu.sync_copy(zero_v, acc.at[pl.ds(base_r + i * 68, 68)])
            return carry
        lax.fori_loop(0, 92, zloop, 0)

        def zloop2(i, carry):
            pltpu.sync_copy(zero_v.at[0], accs.at[pl.ds(base_r + i * 16, 16)])
            return carry
        lax.fori_loop(0, NSTRIPE // 16, zloop2, 0)
        plsc.subcore_barrier()

        rbase = wid * (EPW // BLK)
        lanes = lax.iota(jnp.int32, 16)

        def grp(g, carry):
            gb = rbase + g * GB
            pltpu.sync_copy(src2.at[pl.ds(gb, GB)], src_i)
            pltpu.sync_copy(dst2.at[pl.ds(gb, GB)], dst_i)

            def batch(q, carry2):
                bb = q * NB_L

                def sc16(t, carry3):
                    r = t // (BLK // 16)
                    w = (t % (BLK // 16)) * 16
                    b = bb + r
                    sidx[r, pl.ds(w, 16)] = src_i[b, pl.ds(w, 16)] * 16
                    didx[r, pl.ds(w, 16)] = dst_i[b, pl.ds(w, 16)] * 16
                    e = (gb + b) * BLK + w + lanes
                    cidx[r, pl.ds(w, 16)] = \
                        jnp.minimum(e, E_RAW - 1) * 16
                    return carry3
                lax.fori_loop(0, NB_L * (BLK // 16), sc16, 0)

                handles = []
                for r in range(NB_L):
                    b = bb + r
                    handles.append((
                        pltpu.async_copy(tab.at[src_i.at[b]],
                                         rows_v.at[pl.ds(r * BLK, BLK)],
                                         semg),
                        pltpu.async_copy(s1d.at[sidx.at[r]],
                                         s_v.at[pl.ds(r * BLK, BLK)], semg),
                        pltpu.async_copy(d1d.at[didx.at[r]],
                                         d_v.at[pl.ds(r * BLK, BLK)], semg),
                        pltpu.async_copy(c1d.at[cidx.at[r]],
                                         c_v.at[pl.ds(r * BLK, BLK)], semg),
                    ))
                scat = []
                for r in range(NB_L):
                    b = bb + r
                    for hh in handles[r]:
                        hh.wait()

                    def edge16(jj, carry3, r=r):
                        o = r * BLK + jj * 16
                        logit = s_v[pl.ds(o, 16)] + d_v[pl.ds(o, 16)] \
                            + c_v[pl.ds(o, 16)]
                        logit = jnp.where(logit > 0, logit, 0.2 * logit)
                        exv = jnp.exp(logit)
                        ex_v[pl.ds(o, 16)] = exv
                        for k in range(16):
                            j = o + k
                            exb = _shuf(exv, (lanes * 0 + k).reshape(16, 1))
                            rows_v[j] = rows_v[j] * exb
                        return carry3
                    lax.fori_loop(0, BLK // 16, edge16, 0)

                    scat.append(pltpu.async_copy(
                        rows_v.at[pl.ds(r * BLK, BLK)],
                        acc.at[dst_i.at[b]], sems, add=True))
                    scat.append(pltpu.async_copy(
                        ex_v.at[pl.ds(r * BLK, BLK)],
                        accs.at[dst_i.at[b]], sems, add=True))
                for hh in scat:
                    hh.wait()
                return carry2
            lax.fori_loop(0, GB // NB_L, batch, 0)
            return carry
        lax.fori_loop(0, NG, grp, 0)

        plsc.subcore_barrier()

        def dump(i, carry):
            pltpu.sync_copy(acc.at[pl.ds(base_r + i * 68, 68)],
                            out16.at[cid, pl.ds(base_r + i * 68, 68)])
            return carry
        lax.fori_loop(0, 92, dump, 0)
        pltpu.sync_copy(accs.at[pl.ds(base_r, NSTRIPE)],
                        outS.at[pl.ds(cid * NP + base_r, NSTRIPE)])

    return layer_pass


def _make_final_pass():
    mesh = plsc.VectorSubcoreMesh(core_axis_name="c", subcore_axis_name="s")

    @functools.partial(
        pl.kernel,
        compiler_params=pltpu.CompilerParams(use_tc_tiling_on_sc=False),
        out_type=jax.ShapeDtypeStruct((EP,), jnp.float32),
        mesh=mesh,
        scratch_types=dict(
            src_i=pltpu.VMEM((GB, BLK), jnp.int32),
            dst_i=pltpu.VMEM((GB, BLK), jnp.int32),
            gidx=pltpu.VMEM((NB_F, BLK), jnp.int32),
            u_v=pltpu.VMEM((NB_F * BLK, 16), jnp.float32),
            v_v=pltpu.VMEM((NB_F * BLK, 16), jnp.float32),
            g_v=pltpu.VMEM((NB_F * BLK, 16), jnp.float32),
            p_v=pltpu.VMEM((GB * BLK,), jnp.float32),
            w_v=pltpu.VMEM((32,), jnp.float32),
            semg=pltpu.SemaphoreType.DMA,
            semp=pltpu.SemaphoreType.DMA,
        ),
    )
    def final_pass(src2, dst2, gtab, utab, vtab, w2_h, b2_h, outp,
                   src_i, dst_i, gidx, u_v, v_v, g_v, p_v, w_v, semg, semp):
        cid = lax.axis_index("c")
        sid = lax.axis_index("s")
        wid = cid * 16 + sid

        pltpu.sync_copy(w2_h, w_v.at[pl.ds(0, 16)])
        pltpu.sync_copy(b2_h, w_v.at[pl.ds(16, 16)])
        w2_v = w_v[pl.ds(0, 16)]
        b2_v = w_v[pl.ds(16, 16)]

        rbase = wid * (EPW // BLK)
        lanes = lax.iota(jnp.int32, 16)

        def grp(g, carry):
            gb = rbase + g * GB
            hp = pltpu.async_copy(src2.at[pl.ds(gb, GB)], src_i, semg)
            hq = pltpu.async_copy(dst2.at[pl.ds(gb, GB)], dst_i, semg)
            hp.wait()
            hq.wait()

            def batch(q, carry2):
                bb = q * NB_F

                def gi16(t, carry3):
                    r = t // (BLK // 16)
                    w = (t % (BLK // 16)) * 16
                    e = (gb + bb + r) * BLK + w + lanes
                    gidx[r, pl.ds(w, 16)] = jnp.minimum(e, E_RAW - 1)
                    return carry3
                lax.fori_loop(0, NB_F * (BLK // 16), gi16, 0)

                handles = []
                for r in range(NB_F):
                    b = bb + r
                    handles.append((
                        pltpu.async_copy(utab.at[src_i.at[b]],
                                         u_v.at[pl.ds(r * BLK, BLK)], semg),
                        pltpu.async_copy(vtab.at[dst_i.at[b]],
                                         v_v.at[pl.ds(r * BLK, BLK)], semg),
                        pltpu.async_copy(gtab.at[gidx.at[r]],
                                         g_v.at[pl.ds(r * BLK, BLK)], semg),
                    ))
                for r in range(NB_F):
                    for hh in handles[r]:
                        hh.wait()

                    def edge16(jj, carry3, r=r):
                        o = r * BLK + jj * 16
                        tacc = jnp.zeros((16,), jnp.float32)
                        for k in range(16):
                            j = o + k
                            h = u_v[j] + v_v[j] + g_v[j]
                            h = jnp.maximum(h, 0.0)
                            tb = _sumall(h * w2_v)
                            tacc = jnp.where(lanes == k, tb, tacc)
                        t = tacc + b2_v
                        p_v[pl.ds((bb + r) * BLK + jj * 16, 16)] = \
                            1.0 / (1.0 + jnp.exp(-t))
                        return carry3
                    lax.fori_loop(0, BLK // 16, edge16, 0)
                return carry2
            lax.fori_loop(0, GB // NB_F, batch, 0)

            pltpu.async_copy(p_v, outp.at[pl.ds(gb * BLK, GB * BLK)],
                             semp).wait()
            return carry
        lax.fori_loop(0, NG, grp, 0)

    return final_pass


def kernel(edge_index, node_emb, edge_emb, W_0, asrc_0, adst_0, We_0, aedge_0,
           b_0, W_1, asrc_1, adst_1, We_1, aedge_1, b_1, mlp_W1, mlp_b1,
           mlp_W2, mlp_b2):
    f32 = jnp.float32
    src = edge_index[0]
    dst = edge_index[1]
    pad_e = EP - E_RAW
    pad_ar = np.arange(pad_e, dtype=np.int32)
    src_p = jnp.concatenate([src, jnp.asarray(pad_ar % N_RAW)])
    dst_p = jnp.concatenate([dst, jnp.asarray(N_RAW + pad_ar % (NP - N_RAW))])
    src2 = src_p.reshape(EP // BLK, BLK)
    dst2 = dst_p.reshape(EP // BLK, BLK)

    ee8 = edge_emb.astype(f32).reshape(ER, 128)

    ne_p = jnp.pad(node_emb.astype(f32), ((0, NP - N_RAW), (0, 0)))
    ne8 = ne_p.reshape(NP // 8, 8 * D_IN)

    eye8 = jnp.eye(8, dtype=f32)
    ones16 = jnp.ones((1, 16), f32)

    def rep(vec):
        return vec.astype(f32)[:, None] @ ones16

    K_xp = jnp.kron(eye8, W_0.astype(f32))
    K_d = jnp.kron(eye8, W_0.astype(f32) @ rep(adst_0))
    K_s = jnp.kron(eye8, W_0.astype(f32) @ rep(asrc_0))
    xp0_8, d0_8, s0_8 = _tc_proj3(ne8, K_xp, K_d, K_s, 4)
    tab0 = xp0_8.reshape(NP, 16)
    d0_1d = d0_8.reshape(NP * 16)
    s0_1d = s0_8.reshape(NP * 16)

    Wc = mlp_W1[2 * H:3 * H, :].astype(f32)
    BD_g = jnp.kron(eye8, Wc)
    BD_c0 = jnp.kron(eye8, rep(We_0 @ aedge_0))
    BD_c1 = jnp.kron(eye8, rep(We_1 @ aedge_1))
    g8, c0_8, c1_8 = _tc_proj3(ee8, BD_g, BD_c0, BD_c1, 25)
    gtab = g8.reshape(E_RAW, 16)
    c0_1d = c0_8.reshape(E_RAW * 16)
    c1_1d = c1_8.reshape(E_RAW * 16)

    lp = _make_layer_pass()
    out16_0, outS_0 = lp(src2, dst2, tab0, s0_1d, d0_1d, c0_1d)

    num2_0 = out16_0.reshape(2, NP // 8, 128)
    den_0 = outS_0[:NP] + outS_0[NP:]
    den16_0 = jnp.repeat(den_0, H).reshape(NP // 8, 128)
    b0tile = jnp.tile(b_0.astype(f32), 8)[None, :]
    W1f = W_1.astype(f32)
    BD8_W1 = jnp.kron(eye8, W1f)
    BD8_W1Md = jnp.kron(eye8, W1f @ rep(adst_1))
    BD8_W1Ms = jnp.kron(eye8, W1f @ rep(asrc_1))
    xp1_8, d1_8, s1_8 = _tc_node_activation(num2_0, den16_0, b0tile,
                                            BD8_W1, BD8_W1Md, BD8_W1Ms)
    tab1 = xp1_8.reshape(NP, 16)
    d1_1d = d1_8.reshape(NP * 16)
    s1_1d = s1_8.reshape(NP * 16)

    out16_1, outS_1 = lp(src2, dst2, tab1, s1_1d, d1_1d, c1_1d)

    num2_1 = out16_1.reshape(2, NP // 8, 128)
    den_1 = outS_1[:NP] + outS_1[NP:]
    den16_1 = jnp.repeat(den_1, H).reshape(NP // 8, 128)
    b1gtile = jnp.tile(b_1.astype(f32), 8)[None, :]
    BD8_Wa = jnp.kron(eye8, mlp_W1[:H, :].astype(f32))
    BD8_Wb = jnp.kron(eye8, mlp_W1[H:2 * H, :].astype(f32))
    b1tile = jnp.tile(mlp_b1.astype(f32), 8)[None, :]
    u8, v8 = _tc_uv(num2_1, den16_1, b1gtile, BD8_Wa, BD8_Wb, b1tile)
    utab = u8.reshape(NP, 16)
    vtab = v8.reshape(NP, 16)

    fp = _make_final_pass()
    w2_h = mlp_W2[:, 0].astype(f32)
    b2_h = jnp.full((16,), mlp_b2[0], f32)
    p = fp(src2, dst2, gtab, utab, vtab, w2_h, b2_h)
    return p[:E_RAW]


# speedup vs baseline: 48.3829x; 1.0570x over previous
import functools

import numpy as np
import jax
import jax.numpy as jnp
from jax import lax
from jax.experimental import pallas as pl
from jax.experimental.pallas import tpu as pltpu
from jax.experimental.pallas import tpu_sc as plsc

N_RAW = 100000
E_RAW = 1600000
D_IN = 64
H = 16

NP = 100096
NW = 32
BLK = 128
GB = 16
NG = 25
EPW = NG * GB * BLK
EP = NW * EPW
NSTRIPE = NP // 16
NB_L = 4
NB_F = 8
ER = E_RAW * H // 128


def _proj3_body(x_ref, wa_ref, wb_ref, wc_ref, a_ref, b_ref, c_ref):
    x = x_ref[...]
    a_ref[...] = jnp.dot(x, wa_ref[...], preferred_element_type=jnp.float32)
    b_ref[...] = jnp.dot(x, wb_ref[...], preferred_element_type=jnp.float32)
    c_ref[...] = jnp.dot(x, wc_ref[...], preferred_element_type=jnp.float32)


def _tc_proj3(x, wa, wb, wc, grid):
    rows = x.shape[0]
    br = rows // grid
    out = jax.ShapeDtypeStruct((rows, 128), jnp.float32)
    wspec = pl.BlockSpec(wa.shape, lambda i: (0, 0))
    bspec = pl.BlockSpec((br, 128), lambda i: (i, 0))
    return pl.pallas_call(
        _proj3_body,
        out_shape=(out, out, out),
        grid=(grid,),
        in_specs=[pl.BlockSpec((br, x.shape[1]), lambda i: (i, 0)),
                  wspec, wspec, wspec],
        out_specs=(bspec, bspec, bspec),
    )(x, wa, wb, wc)


def _nodeact_body(num_ref, den_ref, b_ref, w1_ref, wd_ref, ws_ref,
                  xp_ref, d_ref, s_ref):
    num = num_ref[0] + num_ref[1]
    x = jax.nn.gelu(num / (den_ref[...] + 1e-16) + b_ref[...])
    xp_ref[...] = jnp.dot(x, w1_ref[...], preferred_element_type=jnp.float32)
    d_ref[...] = jnp.dot(x, wd_ref[...], preferred_element_type=jnp.float32)
    s_ref[...] = jnp.dot(x, ws_ref[...], preferred_element_type=jnp.float32)


def _tc_node_activation(num2, den16, btile, bd_w, bd_wd, bd_ws):
    rows = NP // 8
    br = rows // 4
    out = jax.ShapeDtypeStruct((rows, 128), jnp.float32)
    wspec = pl.BlockSpec((128, 128), lambda i: (0, 0))
    bspec = pl.BlockSpec((br, 128), lambda i: (i, 0))
    return pl.pallas_call(
        _nodeact_body,
        out_shape=(out, out, out),
        grid=(4,),
        in_specs=[pl.BlockSpec((2, br, 128), lambda i: (0, i, 0)),
                  bspec,
                  pl.BlockSpec((1, 128), lambda i: (0, 0)),
                  wspec, wspec, wspec],
        out_specs=(bspec, bspec, bspec),
    )(num2, den16, btile, bd_w, bd_wd, bd_ws)


def _uv_body(num_ref, den_ref, b_ref, wu_ref, wv_ref, bu_ref, u_ref, v_ref):
    num = num_ref[0] + num_ref[1]
    x = jax.nn.gelu(num / (den_ref[...] + 1e-16) + b_ref[...])
    u_ref[...] = jnp.dot(x, wu_ref[...],
                         preferred_element_type=jnp.float32) + bu_ref[...]
    v_ref[...] = jnp.dot(x, wv_ref[...], preferred_element_type=jnp.float32)


def _tc_uv(num2, den16, btile, bd_wu, bd_wv, b1tile):
    rows = NP // 8
    br = rows // 4
    out = jax.ShapeDtypeStruct((rows, 128), jnp.float32)
    wspec = pl.BlockSpec((128, 128), lambda i: (0, 0))
    bspec = pl.BlockSpec((br, 128), lambda i: (i, 0))
    return pl.pallas_call(
        _uv_body,
        out_shape=(out, out),
        grid=(4,),
        in_specs=[pl.BlockSpec((2, br, 128), lambda i: (0, i, 0)),
                  bspec,
                  pl.BlockSpec((1, 128), lambda i: (0, 0)),
                  wspec, wspec,
                  pl.BlockSpec((1, 128), lambda i: (0, 0))],
        out_specs=(bspec, bspec),
    )(num2, den16, btile, bd_wu, bd_wv, b1tile)


_GDN = lax.GatherDimensionNumbers(offset_dims=(), collapsed_slice_dims=(0,),
                                  start_index_map=(0,))


def _shuf(v, perm):
    return lax.gather(v, perm, _GDN, slice_sizes=(1,),
                      mode=lax.GatherScatterMode.PROMISE_IN_BOUNDS)


def _sumall(v):
    lanes = lax.iota(jnp.int32, 16)
    for s in (8, 4, 2, 1):
        v = v + _shuf(v, (lanes ^ s).reshape(16, 1))
    return v


def _make_layer_pass():
    mesh = plsc.VectorSubcoreMesh(core_axis_name="c", subcore_axis_name="s")

    @functools.partial(
        pl.kernel,
        compiler_params=pltpu.CompilerParams(use_tc_tiling_on_sc=False),
        out_type=(
            jax.ShapeDtypeStruct((2, NP, 16), jnp.float32),
            jax.ShapeDtypeStruct((2 * NP,), jnp.float32),
        ),
        mesh=mesh,
        scratch_types=dict(
            src_i=pltpu.VMEM((GB, BLK), jnp.int32),
            dst_i=pltpu.VMEM((GB, BLK), jnp.int32),
            sidx=pltpu.VMEM((NB_L, BLK), jnp.int32),
            didx=pltpu.VMEM((NB_L, BLK), jnp.int32),
            cidx=pltpu.VMEM((NB_L, BLK), jnp.int32),
            rows_v=pltpu.VMEM((NB_L * BLK, 16), jnp.float32),
            s_v=pltpu.VMEM((NB_L * BLK,), jnp.float32),
            d_v=pltpu.VMEM((NB_L * BLK,), jnp.float32),
            c_v=pltpu.VMEM((NB_L * BLK,), jnp.float32),
            ex_v=pltpu.VMEM((NB_L * BLK,), jnp.float32),
            zero_v=pltpu.VMEM((272, 16), jnp.float32),
            zero1=pltpu.VMEM((1088,), jnp.float32),
            acc=pltpu.VMEM_SHARED((NP, 16), jnp.float32),
            accs=pltpu.VMEM_SHARED((NP,), jnp.float32),
            semg=pltpu.SemaphoreType.DMA,
            sems=pltpu.SemaphoreType.DMA,
            semz=pltpu.SemaphoreType.DMA,
        ),
    )
    def layer_pass(src2, dst2, tab, s1d, d1d, c1d, out16, outS,
                   src_i, dst_i, sidx, didx, cidx, rows_v, s_v, d_v, c_v,
                   ex_v, zero_v, zero1, acc, accs, semg, sems, semz):
        cid = lax.axis_index("c")
        sid = lax.axis_index("s")
        wid = cid * 16 + sid

        def zv(r, carry):
            zero_v[r] = jnp.zeros((16,), jnp.float32)
            return carry
        lax.fori_loop(0, 272, zv, 0)

        def zv1(r, carry):
            zero1[pl.ds(r * 16, 16)] = jnp.zeros((16,), jnp.float32)
            return carry
        lax.fori_loop(0, 68, zv1, 0)
        base_r = sid * NSTRIPE

        zh = [pltpu.async_copy(zero_v, acc.at[pl.ds(base_r + i * 272, 272)],
                               semz) for i in range(23)]
        zh += [pltpu.async_copy(zero1, accs.at[pl.ds(base_r + i * 1088, 1088)],
                                semz) for i in range(5)]
        zh.append(pltpu.async_copy(zero1.at[pl.ds(0, 816)],
                                   accs.at[pl.ds(base_r + 5440, 816)], semz))
        for hh in zh:
            hh.wait()
        plsc.subcore_barrier()

        rbase = wid * (EPW // BLK)
        lanes = lax.iota(jnp.int32, 16)

        def grp(g, carry):
            gb = rbase + g * GB
            pltpu.sync_copy(src2.at[pl.ds(gb, GB)], src_i)
            pltpu.sync_copy(dst2.at[pl.ds(gb, GB)], dst_i)

            def batch(q, carry2):
                bb = q * NB_L

                def sc16(t, carry3):
                    r = t // (BLK // 16)
                    w = (t % (BLK // 16)) * 16
                    b = bb + r
                    sidx[r, pl.ds(w, 16)] = src_i[b, pl.ds(w, 16)] * 16
                    didx[r, pl.ds(w, 16)] = dst_i[b, pl.ds(w, 16)] * 16
                    e = (gb + b) * BLK + w + lanes
                    cidx[r, pl.ds(w, 16)] = \
                        jnp.minimum(e, E_RAW - 1) * 16
                    return carry3
                lax.fori_loop(0, NB_L * (BLK // 16), sc16, 0)

                handles = []
                for r in range(NB_L):
                    b = bb + r
                    handles.append((
                        pltpu.async_copy(tab.at[src_i.at[b]],
                                         rows_v.at[pl.ds(r * BLK, BLK)],
                                         semg),
                        pltpu.async_copy(s1d.at[sidx.at[r]],
                                         s_v.at[pl.ds(r * BLK, BLK)], semg),
                        pltpu.async_copy(d1d.at[didx.at[r]],
                                         d_v.at[pl.ds(r * BLK, BLK)], semg),
                        pltpu.async_copy(c1d.at[cidx.at[r]],
                                         c_v.at[pl.ds(r * BLK, BLK)], semg),
                    ))
                scat = []
                for r in range(NB_L):
                    b = bb + r
                    for hh in handles[r]:
                        hh.wait()

                    def edge16(jj, carry3, r=r):
                        o = r * BLK + jj * 16
                        logit = s_v[pl.ds(o, 16)] + d_v[pl.ds(o, 16)] \
                            + c_v[pl.ds(o, 16)]
                        logit = jnp.where(logit > 0, logit, 0.2 * logit)
                        exv = jnp.exp(logit)
                        ex_v[pl.ds(o, 16)] = exv
                        for k in range(16):
                            j = o + k
                            exb = _shuf(exv, (lanes * 0 + k).reshape(16, 1))
                            rows_v[j] = rows_v[j] * exb
                        return carry3
                    lax.fori_loop(0, BLK // 16, edge16, 0)

                    scat.append(pltpu.async_copy(
                        rows_v.at[pl.ds(r * BLK, BLK)],
                        acc.at[dst_i.at[b]], sems, add=True))
                    scat.append(pltpu.async_copy(
                        ex_v.at[pl.ds(r * BLK, BLK)],
                        accs.at[dst_i.at[b]], sems, add=True))
                for hh in scat:
                    hh.wait()
                return carry2
            lax.fori_loop(0, GB // NB_L, batch, 0)
            return carry
        lax.fori_loop(0, NG, grp, 0)

        plsc.subcore_barrier()

        dh = [pltpu.async_copy(acc.at[pl.ds(base_r + i * 272, 272)],
                               out16.at[cid, pl.ds(base_r + i * 272, 272)],
                               semz) for i in range(23)]
        dh.append(pltpu.async_copy(accs.at[pl.ds(base_r, NSTRIPE)],
                                   outS.at[pl.ds(cid * NP + base_r, NSTRIPE)],
                                   semz))
        for hh in dh:
            hh.wait()

    return layer_pass


def _make_final_pass():
    mesh = plsc.VectorSubcoreMesh(core_axis_name="c", subcore_axis_name="s")

    @functools.partial(
        pl.kernel,
        compiler_params=pltpu.CompilerParams(use_tc_tiling_on_sc=False),
        out_type=jax.ShapeDtypeStruct((EP,), jnp.float32),
        mesh=mesh,
        scratch_types=dict(
            src_i=pltpu.VMEM((GB, BLK), jnp.int32),
            dst_i=pltpu.VMEM((GB, BLK), jnp.int32),
            gidx=pltpu.VMEM((NB_F, BLK), jnp.int32),
            u_v=pltpu.VMEM((NB_F * BLK, 16), jnp.float32),
            v_v=pltpu.VMEM((NB_F * BLK, 16), jnp.float32),
            g_v=pltpu.VMEM((NB_F * BLK, 16), jnp.float32),
            p_v=pltpu.VMEM((GB * BLK,), jnp.float32),
            w_v=pltpu.VMEM((32,), jnp.float32),
            semg=pltpu.SemaphoreType.DMA,
            semp=pltpu.SemaphoreType.DMA,
        ),
    )
    def final_pass(src2, dst2, gtab, utab, vtab, w2_h, b2_h, outp,
                   src_i, dst_i, gidx, u_v, v_v, g_v, p_v, w_v, semg, semp):
        cid = lax.axis_index("c")
        sid = lax.axis_index("s")
        wid = cid * 16 + sid

        pltpu.sync_copy(w2_h, w_v.at[pl.ds(0, 16)])
        pltpu.sync_copy(b2_h, w_v.at[pl.ds(16, 16)])
        w2_v = w_v[pl.ds(0, 16)]
        b2_v = w_v[pl.ds(16, 16)]

        rbase = wid * (EPW // BLK)
        lanes = lax.iota(jnp.int32, 16)

        def grp(g, carry):
            gb = rbase + g * GB
            hp = pltpu.async_copy(src2.at[pl.ds(gb, GB)], src_i, semg)
            hq = pltpu.async_copy(dst2.at[pl.ds(gb, GB)], dst_i, semg)
            hp.wait()
            hq.wait()

            def batch(q, carry2):
                bb = q * NB_F

                def gi16(t, carry3):
                    r = t // (BLK // 16)
                    w = (t % (BLK // 16)) * 16
                    e = (gb + bb + r) * BLK + w + lanes
                    gidx[r, pl.ds(w, 16)] = jnp.minimum(e, E_RAW - 1)
                    return carry3
                lax.fori_loop(0, NB_F * (BLK // 16), gi16, 0)

                handles = []
                for r in range(NB_F):
                    b = bb + r
                    handles.append((
                        pltpu.async_copy(utab.at[src_i.at[b]],
                                         u_v.at[pl.ds(r * BLK, BLK)], semg),
                        pltpu.async_copy(vtab.at[dst_i.at[b]],
                                         v_v.at[pl.ds(r * BLK, BLK)], semg),
                        pltpu.async_copy(gtab.at[gidx.at[r]],
                                         g_v.at[pl.ds(r * BLK, BLK)], semg),
                    ))
                for r in range(NB_F):
                    for hh in handles[r]:
                        hh.wait()

                    def edge16(jj, carry3, r=r):
                        o = r * BLK + jj * 16
                        tacc = jnp.zeros((16,), jnp.float32)
                        for k in range(16):
                            j = o + k
                            h = u_v[j] + v_v[j] + g_v[j]
                            h = jnp.maximum(h, 0.0)
                            tb = _sumall(h * w2_v)
                            tacc = jnp.where(lanes == k, tb, tacc)
                        t = tacc + b2_v
                        p_v[pl.ds((bb + r) * BLK + jj * 16, 16)] = \
                            1.0 / (1.0 + jnp.exp(-t))
                        return carry3
                    lax.fori_loop(0, BLK // 16, edge16, 0)
                return carry2
            lax.fori_loop(0, GB // NB_F, batch, 0)

            pltpu.async_copy(p_v, outp.at[pl.ds(gb * BLK, GB * BLK)],
                             semp).wait()
            return carry
        lax.fori_loop(0, NG, grp, 0)

    return final_pass


def kernel(edge_index, node_emb, edge_emb, W_0, asrc_0, adst_0, We_0, aedge_0,
           b_0, W_1, asrc_1, adst_1, We_1, aedge_1, b_1, mlp_W1, mlp_b1,
           mlp_W2, mlp_b2):
    f32 = jnp.float32
    src = edge_index[0]
    dst = edge_index[1]
    pad_e = EP - E_RAW
    pad_ar = np.arange(pad_e, dtype=np.int32)
    src_p = jnp.concatenate([src, jnp.asarray(pad_ar % N_RAW)])
    dst_p = jnp.concatenate([dst, jnp.asarray(N_RAW + pad_ar % (NP - N_RAW))])
    src2 = src_p.reshape(EP // BLK, BLK)
    dst2 = dst_p.reshape(EP // BLK, BLK)

    ee8 = edge_emb.astype(f32).reshape(ER, 128)

    ne_p = jnp.pad(node_emb.astype(f32), ((0, NP - N_RAW), (0, 0)))
    ne8 = ne_p.reshape(NP // 8, 8 * D_IN)

    eye8 = jnp.eye(8, dtype=f32)
    ones16 = jnp.ones((1, 16), f32)

    def rep(vec):
        return vec.astype(f32)[:, None] @ ones16

    K_xp = jnp.kron(eye8, W_0.astype(f32))
    K_d = jnp.kron(eye8, W_0.astype(f32) @ rep(adst_0))
    K_s = jnp.kron(eye8, W_0.astype(f32) @ rep(asrc_0))
    xp0_8, d0_8, s0_8 = _tc_proj3(ne8, K_xp, K_d, K_s, 4)
    tab0 = xp0_8.reshape(NP, 16)
    d0_1d = d0_8.reshape(NP * 16)
    s0_1d = s0_8.reshape(NP * 16)

    Wc = mlp_W1[2 * H:3 * H, :].astype(f32)
    BD_g = jnp.kron(eye8, Wc)
    BD_c0 = jnp.kron(eye8, rep(We_0 @ aedge_0))
    BD_c1 = jnp.kron(eye8, rep(We_1 @ aedge_1))
    g8, c0_8, c1_8 = _tc_proj3(ee8, BD_g, BD_c0, BD_c1, 25)
    gtab = g8.reshape(E_RAW, 16)
    c0_1d = c0_8.reshape(E_RAW * 16)
    c1_1d = c1_8.reshape(E_RAW * 16)

    lp = _make_layer_pass()
    out16_0, outS_0 = lp(src2, dst2, tab0, s0_1d, d0_1d, c0_1d)

    num2_0 = out16_0.reshape(2, NP // 8, 128)
    den_0 = outS_0[:NP] + outS_0[NP:]
    den16_0 = jnp.repeat(den_0, H).reshape(NP // 8, 128)
    b0tile = jnp.tile(b_0.astype(f32), 8)[None, :]
    W1f = W_1.astype(f32)
    BD8_W1 = jnp.kron(eye8, W1f)
    BD8_W1Md = jnp.kron(eye8, W1f @ rep(adst_1))
    BD8_W1Ms = jnp.kron(eye8, W1f @ rep(asrc_1))
    xp1_8, d1_8, s1_8 = _tc_node_activation(num2_0, den16_0, b0tile,
                                            BD8_W1, BD8_W1Md, BD8_W1Ms)
    tab1 = xp1_8.reshape(NP, 16)
    d1_1d = d1_8.reshape(NP * 16)
    s1_1d = s1_8.reshape(NP * 16)

    out16_1, outS_1 = lp(src2, dst2, tab1, s1_1d, d1_1d, c1_1d)

    num2_1 = out16_1.reshape(2, NP // 8, 128)
    den_1 = outS_1[:NP] + outS_1[NP:]
    den16_1 = jnp.repeat(den_1, H).reshape(NP // 8, 128)
    b1gtile = jnp.tile(b_1.astype(f32), 8)[None, :]
    BD8_Wa = jnp.kron(eye8, mlp_W1[:H, :].astype(f32))
    BD8_Wb = jnp.kron(eye8, mlp_W1[H:2 * H, :].astype(f32))
    b1tile = jnp.tile(mlp_b1.astype(f32), 8)[None, :]
    u8, v8 = _tc_uv(num2_1, den16_1, b1gtile, BD8_Wa, BD8_Wb, b1tile)
    utab = u8.reshape(NP, 16)
    vtab = v8.reshape(NP, 16)

    fp = _make_final_pass()
    w2_h = mlp_W2[:, 0].astype(f32)
    b2_h = jnp.full((16,), mlp_b2[0], f32)
    p = fp(src2, dst2, gtab, utab, vtab, w2_h, b2_h)
    return p[:E_RAW]


# speedup vs baseline: 49.0946x; 1.0147x over previous
import functools

import numpy as np
import jax
import jax.numpy as jnp
from jax import lax
from jax.experimental import pallas as pl
from jax.experimental.pallas import tpu as pltpu
from jax.experimental.pallas import tpu_sc as plsc

N_RAW = 100000
E_RAW = 1600000
D_IN = 64
H = 16

NP = 100096
NW = 32
BLK = 128
GB = 16
NG = 25
EPW = NG * GB * BLK
EP = NW * EPW
NSTRIPE = NP // 16
NB_L = 4
NB_F = 16
ER = E_RAW * H // 128


def _proj3_body(x_ref, wa_ref, wb_ref, wc_ref, a_ref, b_ref, c_ref):
    x = x_ref[...]
    a_ref[...] = jnp.dot(x, wa_ref[...], preferred_element_type=jnp.float32)
    b_ref[...] = jnp.dot(x, wb_ref[...], preferred_element_type=jnp.float32)
    c_ref[...] = jnp.dot(x, wc_ref[...], preferred_element_type=jnp.float32)


def _tc_proj3(x, wa, wb, wc, grid):
    rows = x.shape[0]
    br = rows // grid
    out = jax.ShapeDtypeStruct((rows, 128), jnp.float32)
    wspec = pl.BlockSpec(wa.shape, lambda i: (0, 0))
    bspec = pl.BlockSpec((br, 128), lambda i: (i, 0))
    return pl.pallas_call(
        _proj3_body,
        out_shape=(out, out, out),
        grid=(grid,),
        in_specs=[pl.BlockSpec((br, x.shape[1]), lambda i: (i, 0)),
                  wspec, wspec, wspec],
        out_specs=(bspec, bspec, bspec),
    )(x, wa, wb, wc)


def _nodeact_body(num_ref, den_ref, b_ref, w1_ref, wd_ref, ws_ref,
                  xp_ref, d_ref, s_ref):
    num = num_ref[0] + num_ref[1]
    x = jax.nn.gelu(num / (den_ref[...] + 1e-16) + b_ref[...])
    xp_ref[...] = jnp.dot(x, w1_ref[...], preferred_element_type=jnp.float32)
    d_ref[...] = jnp.dot(x, wd_ref[...], preferred_element_type=jnp.float32)
    s_ref[...] = jnp.dot(x, ws_ref[...], preferred_element_type=jnp.float32)


def _tc_node_activation(num2, den16, btile, bd_w, bd_wd, bd_ws):
    rows = NP // 8
    br = rows // 4
    out = jax.ShapeDtypeStruct((rows, 128), jnp.float32)
    wspec = pl.BlockSpec((128, 128), lambda i: (0, 0))
    bspec = pl.BlockSpec((br, 128), lambda i: (i, 0))
    return pl.pallas_call(
        _nodeact_body,
        out_shape=(out, out, out),
        grid=(4,),
        in_specs=[pl.BlockSpec((2, br, 128), lambda i: (0, i, 0)),
                  bspec,
                  pl.BlockSpec((1, 128), lambda i: (0, 0)),
                  wspec, wspec, wspec],
        out_specs=(bspec, bspec, bspec),
    )(num2, den16, btile, bd_w, bd_wd, bd_ws)


def _uv_body(num_ref, den_ref, b_ref, wu_ref, wv_ref, bu_ref, u_ref, v_ref):
    num = num_ref[0] + num_ref[1]
    x = jax.nn.gelu(num / (den_ref[...] + 1e-16) + b_ref[...])
    u_ref[...] = jnp.dot(x, wu_ref[...],
                         preferred_element_type=jnp.float32) + bu_ref[...]
    v_ref[...] = jnp.dot(x, wv_ref[...], preferred_element_type=jnp.float32)


def _tc_uv(num2, den16, btile, bd_wu, bd_wv, b1tile):
    rows = NP // 8
    br = rows // 4
    out = jax.ShapeDtypeStruct((rows, 128), jnp.float32)
    wspec = pl.BlockSpec((128, 128), lambda i: (0, 0))
    bspec = pl.BlockSpec((br, 128), lambda i: (i, 0))
    return pl.pallas_call(
        _uv_body,
        out_shape=(out, out),
        grid=(4,),
        in_specs=[pl.BlockSpec((2, br, 128), lambda i: (0, i, 0)),
                  bspec,
                  pl.BlockSpec((1, 128), lambda i: (0, 0)),
                  wspec, wspec,
                  pl.BlockSpec((1, 128), lambda i: (0, 0))],
        out_specs=(bspec, bspec),
    )(num2, den16, btile, bd_wu, bd_wv, b1tile)


_GDN = lax.GatherDimensionNumbers(offset_dims=(), collapsed_slice_dims=(0,),
                                  start_index_map=(0,))


def _shuf(v, perm):
    return lax.gather(v, perm, _GDN, slice_sizes=(1,),
                      mode=lax.GatherScatterMode.PROMISE_IN_BOUNDS)


def _sumall(v):
    lanes = lax.iota(jnp.int32, 16)
    for s in (8, 4, 2, 1):
        v = v + _shuf(v, (lanes ^ s).reshape(16, 1))
    return v


def _make_layer_pass():
    mesh = plsc.VectorSubcoreMesh(core_axis_name="c", subcore_axis_name="s")

    @functools.partial(
        pl.kernel,
        compiler_params=pltpu.CompilerParams(use_tc_tiling_on_sc=False),
        out_type=(
            jax.ShapeDtypeStruct((2, NP, 16), jnp.float32),
            jax.ShapeDtypeStruct((2 * NP,), jnp.float32),
        ),
        mesh=mesh,
        scratch_types=dict(
            src_i=pltpu.VMEM((GB, BLK), jnp.int32),
            dst_i=pltpu.VMEM((GB, BLK), jnp.int32),
            sidx=pltpu.VMEM((NB_L, BLK), jnp.int32),
            didx=pltpu.VMEM((NB_L, BLK), jnp.int32),
            cidx=pltpu.VMEM((NB_L, BLK), jnp.int32),
            rows_v=pltpu.VMEM((NB_L * BLK, 16), jnp.float32),
            s_v=pltpu.VMEM((NB_L * BLK,), jnp.float32),
            d_v=pltpu.VMEM((NB_L * BLK,), jnp.float32),
            c_v=pltpu.VMEM((NB_L * BLK,), jnp.float32),
            ex_v=pltpu.VMEM((NB_L * BLK,), jnp.float32),
            zero_v=pltpu.VMEM((272, 16), jnp.float32),
            zero1=pltpu.VMEM((1088,), jnp.float32),
            acc=pltpu.VMEM_SHARED((NP, 16), jnp.float32),
            accs=pltpu.VMEM_SHARED((NP,), jnp.float32),
            semg=pltpu.SemaphoreType.DMA,
            sems=pltpu.SemaphoreType.DMA,
            semz=pltpu.SemaphoreType.DMA,
        ),
    )
    def layer_pass(src2, dst2, tab, s1d, d1d, c1d, out16, outS,
                   src_i, dst_i, sidx, didx, cidx, rows_v, s_v, d_v, c_v,
                   ex_v, zero_v, zero1, acc, accs, semg, sems, semz):
        cid = lax.axis_index("c")
        sid = lax.axis_index("s")
        wid = cid * 16 + sid

        def zv(r, carry):
            zero_v[r] = jnp.zeros((16,), jnp.float32)
            return carry
        lax.fori_loop(0, 272, zv, 0)

        def zv1(r, carry):
            zero1[pl.ds(r * 16, 16)] = jnp.zeros((16,), jnp.float32)
            return carry
        lax.fori_loop(0, 68, zv1, 0)
        base_r = sid * NSTRIPE

        zh = [pltpu.async_copy(zero_v, acc.at[pl.ds(base_r + i * 272, 272)],
                               semz) for i in range(23)]
        zh += [pltpu.async_copy(zero1, accs.at[pl.ds(base_r + i * 1088, 1088)],
                                semz) for i in range(5)]
        zh.append(pltpu.async_copy(zero1.at[pl.ds(0, 816)],
                                   accs.at[pl.ds(base_r + 5440, 816)], semz))
        for hh in zh:
            hh.wait()
        plsc.subcore_barrier()

        rbase = wid * (EPW // BLK)
        lanes = lax.iota(jnp.int32, 16)

        def grp(g, carry):
            gb = rbase + g * GB
            pltpu.sync_copy(src2.at[pl.ds(gb, GB)], src_i)
            pltpu.sync_copy(dst2.at[pl.ds(gb, GB)], dst_i)

            def batch(q, carry2):
                bb = q * NB_L

                def sc16(t, carry3):
                    r = t // (BLK // 16)
                    w = (t % (BLK // 16)) * 16
                    b = bb + r
                    sidx[r, pl.ds(w, 16)] = src_i[b, pl.ds(w, 16)] * 16
                    didx[r, pl.ds(w, 16)] = dst_i[b, pl.ds(w, 16)] * 16
                    e = (gb + b) * BLK + w + lanes
                    cidx[r, pl.ds(w, 16)] = \
                        jnp.minimum(e, E_RAW - 1) * 16
                    return carry3
                lax.fori_loop(0, NB_L * (BLK // 16), sc16, 0)

                handles = []
                for r in range(NB_L):
                    b = bb + r
                    handles.append((
                        pltpu.async_copy(tab.at[src_i.at[b]],
                                         rows_v.at[pl.ds(r * BLK, BLK)],
                                         semg),
                        pltpu.async_copy(s1d.at[sidx.at[r]],
                                         s_v.at[pl.ds(r * BLK, BLK)], semg),
                        pltpu.async_copy(d1d.at[didx.at[r]],
                                         d_v.at[pl.ds(r * BLK, BLK)], semg),
                        pltpu.async_copy(c1d.at[cidx.at[r]],
                                         c_v.at[pl.ds(r * BLK, BLK)], semg),
                    ))
                scat = []
                for r in range(NB_L):
                    b = bb + r
                    for hh in handles[r]:
                        hh.wait()

                    def edge16(jj, carry3, r=r):
                        o = r * BLK + jj * 16
                        logit = s_v[pl.ds(o, 16)] + d_v[pl.ds(o, 16)] \
                            + c_v[pl.ds(o, 16)]
                        logit = jnp.where(logit > 0, logit, 0.2 * logit)
                        exv = jnp.exp(logit)
                        ex_v[pl.ds(o, 16)] = exv
                        for k in range(16):
                            j = o + k
                            exb = _shuf(exv, (lanes * 0 + k).reshape(16, 1))
                            rows_v[j] = rows_v[j] * exb
                        return carry3
                    lax.fori_loop(0, BLK // 16, edge16, 0)

                    scat.append(pltpu.async_copy(
                        rows_v.at[pl.ds(r * BLK, BLK)],
                        acc.at[dst_i.at[b]], sems, add=True))
                    scat.append(pltpu.async_copy(
                        ex_v.at[pl.ds(r * BLK, BLK)],
                        accs.at[dst_i.at[b]], sems, add=True))
                for hh in scat:
                    hh.wait()
                return carry2
            lax.fori_loop(0, GB // NB_L, batch, 0)
            return carry
        lax.fori_loop(0, NG, grp, 0)

        plsc.subcore_barrier()

        dh = [pltpu.async_copy(acc.at[pl.ds(base_r + i * 272, 272)],
                               out16.at[cid, pl.ds(base_r + i * 272, 272)],
                               semz) for i in range(23)]
        dh.append(pltpu.async_copy(accs.at[pl.ds(base_r, NSTRIPE)],
                                   outS.at[pl.ds(cid * NP + base_r, NSTRIPE)],
                                   semz))
        for hh in dh:
            hh.wait()

    return layer_pass


def _make_final_pass():
    mesh = plsc.VectorSubcoreMesh(core_axis_name="c", subcore_axis_name="s")

    @functools.partial(
        pl.kernel,
        compiler_params=pltpu.CompilerParams(use_tc_tiling_on_sc=False),
        out_type=jax.ShapeDtypeStruct((EP,), jnp.float32),
        mesh=mesh,
        scratch_types=dict(
            src_i=pltpu.VMEM((GB, BLK), jnp.int32),
            dst_i=pltpu.VMEM((GB, BLK), jnp.int32),
            gidx=pltpu.VMEM((NB_F, BLK), jnp.int32),
            u_v=pltpu.VMEM((NB_F * BLK, 16), jnp.float32),
            v_v=pltpu.VMEM((NB_F * BLK, 16), jnp.float32),
            g_v=pltpu.VMEM((NB_F * BLK, 16), jnp.float32),
            p_v=pltpu.VMEM((GB * BLK,), jnp.float32),
            w_v=pltpu.VMEM((32,), jnp.float32),
            semg=pltpu.SemaphoreType.DMA,
            semp=pltpu.SemaphoreType.DMA,
        ),
    )
    def final_pass(src2, dst2, gtab, utab, vtab, w2_h, b2_h, outp,
                   src_i, dst_i, gidx, u_v, v_v, g_v, p_v, w_v, semg, semp):
        cid = lax.axis_index("c")
        sid = lax.axis_index("s")
        wid = cid * 16 + sid

        pltpu.sync_copy(w2_h, w_v.at[pl.ds(0, 16)])
        pltpu.sync_copy(b2_h, w_v.at[pl.ds(16, 16)])
        w2_v = w_v[pl.ds(0, 16)]
        b2_v = w_v[pl.ds(16, 16)]

        rbase = wid * (EPW // BLK)
        lanes = lax.iota(jnp.int32, 16)

        def grp(g, carry):
            gb = rbase + g * GB
            hp = pltpu.async_copy(src2.at[pl.ds(gb, GB)], src_i, semg)
            hq = pltpu.async_copy(dst2.at[pl.ds(gb, GB)], dst_i, semg)
            hp.wait()
            hq.wait()

            def batch(q, carry2):
                bb = q * NB_F

                def gi16(t, carry3):
                    r = t // (BLK // 16)
                    w = (t % (BLK // 16)) * 16
                    e = (gb + bb + r) * BLK + w + lanes
                    gidx[r, pl.ds(w, 16)] = jnp.minimum(e, E_RAW - 1)
                    return carry3
                lax.fori_loop(0, NB_F * (BLK // 16), gi16, 0)

                handles = []
                for r in range(NB_F):
                    b = bb + r
                    handles.append((
                        pltpu.async_copy(utab.at[src_i.at[b]],
                                         u_v.at[pl.ds(r * BLK, BLK)], semg),
                        pltpu.async_copy(vtab.at[dst_i.at[b]],
                                         v_v.at[pl.ds(r * BLK, BLK)], semg),
                        pltpu.async_copy(gtab.at[gidx.at[r]],
                                         g_v.at[pl.ds(r * BLK, BLK)], semg),
                    ))
                for r in range(NB_F):
                    for hh in handles[r]:
                        hh.wait()

                    def edge16(jj, carry3, r=r):
                        o = r * BLK + jj * 16
                        tacc = jnp.zeros((16,), jnp.float32)
                        for k in range(16):
                            j = o + k
                            h = u_v[j] + v_v[j] + g_v[j]
                            h = jnp.maximum(h, 0.0)
                            tb = _sumall(h * w2_v)
                            tacc = jnp.where(lanes == k, tb, tacc)
                        t = tacc + b2_v
                        p_v[pl.ds((bb + r) * BLK + jj * 16, 16)] = \
                            1.0 / (1.0 + jnp.exp(-t))
                        return carry3
                    lax.fori_loop(0, BLK // 16, edge16, 0)
                return carry2
            lax.fori_loop(0, GB // NB_F, batch, 0)

            pltpu.async_copy(p_v, outp.at[pl.ds(gb * BLK, GB * BLK)],
                             semp).wait()
            return carry
        lax.fori_loop(0, NG, grp, 0)

    return final_pass


def kernel(edge_index, node_emb, edge_emb, W_0, asrc_0, adst_0, We_0, aedge_0,
           b_0, W_1, asrc_1, adst_1, We_1, aedge_1, b_1, mlp_W1, mlp_b1,
           mlp_W2, mlp_b2):
    f32 = jnp.float32
    src = edge_index[0]
    dst = edge_index[1]
    pad_e = EP - E_RAW
    pad_ar = np.arange(pad_e, dtype=np.int32)
    src_p = jnp.concatenate([src, jnp.asarray(pad_ar % N_RAW)])
    dst_p = jnp.concatenate([dst, jnp.asarray(N_RAW + pad_ar % (NP - N_RAW))])
    src2 = src_p.reshape(EP // BLK, BLK)
    dst2 = dst_p.reshape(EP // BLK, BLK)

    ee8 = edge_emb.astype(f32).reshape(ER, 128)

    ne_p = jnp.pad(node_emb.astype(f32), ((0, NP - N_RAW), (0, 0)))
    ne8 = ne_p.reshape(NP // 8, 8 * D_IN)

    eye8 = jnp.eye(8, dtype=f32)
    ones16 = jnp.ones((1, 16), f32)

    def rep(vec):
        return vec.astype(f32)[:, None] @ ones16

    K_xp = jnp.kron(eye8, W_0.astype(f32))
    K_d = jnp.kron(eye8, W_0.astype(f32) @ rep(adst_0))
    K_s = jnp.kron(eye8, W_0.astype(f32) @ rep(asrc_0))
    xp0_8, d0_8, s0_8 = _tc_proj3(ne8, K_xp, K_d, K_s, 4)
    tab0 = xp0_8.reshape(NP, 16)
    d0_1d = d0_8.reshape(NP * 16)
    s0_1d = s0_8.reshape(NP * 16)

    Wc = mlp_W1[2 * H:3 * H, :].astype(f32)
    BD_g = jnp.kron(eye8, Wc)
    BD_c0 = jnp.kron(eye8, rep(We_0 @ aedge_0))
    BD_c1 = jnp.kron(eye8, rep(We_1 @ aedge_1))
    g8, c0_8, c1_8 = _tc_proj3(ee8, BD_g, BD_c0, BD_c1, 25)
    gtab = g8.reshape(E_RAW, 16)
    c0_1d = c0_8.reshape(E_RAW * 16)
    c1_1d = c1_8.reshape(E_RAW * 16)

    lp = _make_layer_pass()
    out16_0, outS_0 = lp(src2, dst2, tab0, s0_1d, d0_1d, c0_1d)

    num2_0 = out16_0.reshape(2, NP // 8, 128)
    den_0 = outS_0[:NP] + outS_0[NP:]
    den16_0 = jnp.repeat(den_0, H).reshape(NP // 8, 128)
    b0tile = jnp.tile(b_0.astype(f32), 8)[None, :]
    W1f = W_1.astype(f32)
    BD8_W1 = jnp.kron(eye8, W1f)
    BD8_W1Md = jnp.kron(eye8, W1f @ rep(adst_1))
    BD8_W1Ms = jnp.kron(eye8, W1f @ rep(asrc_1))
    xp1_8, d1_8, s1_8 = _tc_node_activation(num2_0, den16_0, b0tile,
                                            BD8_W1, BD8_W1Md, BD8_W1Ms)
    tab1 = xp1_8.reshape(NP, 16)
    d1_1d = d1_8.reshape(NP * 16)
    s1_1d = s1_8.reshape(NP * 16)

    out16_1, outS_1 = lp(src2, dst2, tab1, s1_1d, d1_1d, c1_1d)

    num2_1 = out16_1.reshape(2, NP // 8, 128)
    den_1 = outS_1[:NP] + outS_1[NP:]
    den16_1 = jnp.repeat(den_1, H).reshape(NP // 8, 128)
    b1gtile = jnp.tile(b_1.astype(f32), 8)[None, :]
    BD8_Wa = jnp.kron(eye8, mlp_W1[:H, :].astype(f32))
    BD8_Wb = jnp.kron(eye8, mlp_W1[H:2 * H, :].astype(f32))
    b1tile = jnp.tile(mlp_b1.astype(f32), 8)[None, :]
    u8, v8 = _tc_uv(num2_1, den16_1, b1gtile, BD8_Wa, BD8_Wb, b1tile)
    utab = u8.reshape(NP, 16)
    vtab = v8.reshape(NP, 16)

    fp = _make_final_pass()
    w2_h = mlp_W2[:, 0].astype(f32)
    b2_h = jnp.full((16,), mlp_b2[0], f32)
    p = fp(src2, dst2, gtab, utab, vtab, w2_h, b2_h)
    return p[:E_RAW]
